# baseline (speedup 1.0000x reference)
import functools

import jax
import jax.numpy as jnp
from jax import lax
from jax.experimental import pallas as pl
from jax.experimental.pallas import tpu as pltpu
from jax.experimental.pallas import tpu_sc as plsc

_BATCH = 16384
_FIELDS = 26
_FEAT = 32
_PF = 40
_BB = _BATCH // 128
_NQ = _FIELDS * _BB
_NW = 32
_QPW = _NQ // _NW
_LPW = _QPW * 128
_CQ = 2
_CL = _CQ * 128
_NCH = _QPW // _CQ

_VOC = 1000000
_VOCP = 1000064
_NBLK = _VOCP // 128
_FULLB = _NBLK - 1
_SB = 4
_SBL = _SB * 128
_NSB = _FULLB // _SB
_SPW = _NSB // _NW
_SXTRA = _NSB - _SPW * _NW
_SCR = _VOCP * _PF


def _table_body(tt_hbm, tail_hbm, scr_hbm, in0, in1, in2, ob0, ob1, ob2,
                i0, i1, i2s, o0, o1, o2):
    c = lax.axis_index("c")
    s = lax.axis_index("s")
    wid = s * 2 + c
    start = wid * _SPW + jnp.minimum(wid, _SXTRA)
    nsb = _SPW + jnp.where(wid < _SXTRA, 1, 0)

    ibuf = (in0, in1, in2)
    obuf = (ob0, ob1, ob2)
    isem = (i0, i1, i2s)
    osem = (o0, o1, o2)
    iota16 = lax.iota(jnp.int32, 16)
    avecs = tuple((iota16 + v0) * _PF for v0 in range(0, _SBL, 16))

    def fetch(si, b):
        return pltpu.async_copy(
            tt_hbm.at[:, pl.ds((start + si) * _SBL, _SBL)], ibuf[b], isem[b])

    for p in range(2):
        fetch(p, p)

    def body(it, carry):
        for bp in range(3):
            si = 3 * it + bp

            @pl.when(si < nsb)
            def _():
                pltpu.make_async_copy(
                    tt_hbm.at[:, pl.ds(0, _SBL)], ibuf[bp], isem[bp]).wait()

                @pl.when(si + 2 < nsb)
                def _():
                    fetch(si + 2, (bp + 2) % 3)

                @pl.when(si >= 3)
                def _():
                    pltpu.make_async_copy(
                        obuf[bp], scr_hbm.at[pl.ds(0, _SBL * _PF)],
                        osem[bp]).wait()

                for f in range(_FEAT):
                    for v4 in range(0, _SBL // 16, 4):
                        vals = [ibuf[bp][f, pl.ds((v4 + u) * 16, 16)]
                                for u in range(4)]
                        for u in range(4):
                            plsc.store_scatter(obuf[bp], [avecs[v4 + u] + f],
                                               vals[u])

                pltpu.async_copy(
                    obuf[bp],
                    scr_hbm.at[pl.ds((start + si) * _SBL * _PF, _SBL * _PF)],
                    osem[bp])
        return carry

    lax.fori_loop(0, (_SPW + 3) // 3, body, None)
    for bp in range(3):
        pltpu.make_async_copy(
            obuf[bp], scr_hbm.at[pl.ds(0, _SBL * _PF)], osem[bp]).wait()

    @pl.when(wid == _NW - 1)
    def _():
        pltpu.sync_copy(tail_hbm, in0.at[:, pl.ds(0, 128)])
        for f in range(_FEAT):
            for u in range(4):
                vals = in0[f, pl.ds(u * 16, 16)]
                plsc.store_scatter(ob0, [avecs[u] + f], vals)
        pltpu.sync_copy(ob0.at[pl.ds(0, 64 * _PF)],
                        scr_hbm.at[pl.ds(_FULLB * 128 * _PF, 64 * _PF)])


_table_call = functools.partial(
    pl.kernel,
    out_type=jax.ShapeDtypeStruct((_SCR,), jnp.float32),
    mesh=plsc.VectorSubcoreMesh(core_axis_name="c", subcore_axis_name="s"),
    scratch_types=[
        pltpu.VMEM((_FEAT, _SBL), jnp.float32),
        pltpu.VMEM((_FEAT, _SBL), jnp.float32),
        pltpu.VMEM((_FEAT, _SBL), jnp.float32),
        pltpu.VMEM((_SBL * _PF,), jnp.float32),
        pltpu.VMEM((_SBL * _PF,), jnp.float32),
        pltpu.VMEM((_SBL * _PF,), jnp.float32),
        pltpu.SemaphoreType.DMA,
        pltpu.SemaphoreType.DMA,
        pltpu.SemaphoreType.DMA,
        pltpu.SemaphoreType.DMA,
        pltpu.SemaphoreType.DMA,
        pltpu.SemaphoreType.DMA,
    ],
    compiler_params=pltpu.CompilerParams(
        use_tc_tiling_on_sc=True, needs_layout_passes=False),
)(_table_body)


def _embed_body(idx_hbm, table_hbm, out_hbm, idx_v, rows0, rows1, rows2,
                t0, t1, t2, g0, g1, g2, o0, o1, o2):
    c = lax.axis_index("c")
    s = lax.axis_index("s")
    wid = s * 2 + c
    q0 = wid * _QPW
    pltpu.sync_copy(idx_hbm.at[pl.ds(wid * _LPW, _LPW)], idx_v)

    rows = (rows0, rows1, rows2)
    tbuf = (t0, t1, t2)
    gsem = (g0, g1, g2)
    osem = (o0, o1, o2)
    iota16 = lax.iota(jnp.int32, 16)
    fvecs = tuple(iota16 * 0 + f for f in range(_FEAT))

    def gather(ci, b):
        return pltpu.async_copy(
            table_hbm.at[idx_v.at[pl.ds(ci * _CL, _CL)]], rows[b], gsem[b])

    for p in range(2):
        gather(p, p)

    def step(it, carry):
        for bp in range(3):
            ci = 3 * it + bp

            @pl.when(ci < _NCH)
            def _():
                pltpu.make_async_copy(
                    table_hbm.at[idx_v.at[pl.ds(0, _CL)]], rows[bp],
                    gsem[bp]).wait()

                @pl.when(ci + 2 < _NCH)
                def _():
                    gather(ci + 2, (bp + 2) % 3)

                @pl.when(ci >= 3)
                def _():
                    pltpu.make_async_copy(
                        tbuf[bp], out_hbm.at[0, :, pl.ds(0, _CQ)],
                        osem[bp]).wait()

                for dq in range(_CQ):
                    for j in range(8):
                        kvec = iota16 + (dq * 128 + j * 16)
                        for f0 in range(0, _FEAT, 4):
                            vals = [plsc.load_gather(rows[bp],
                                                     [kvec, fvecs[f0 + u]])
                                    for u in range(4)]
                            for u in range(4):
                                f = f0 + u
                                tbuf[bp][f // 8, dq, f % 8,
                                         pl.ds(j * 16, 16)] = vals[u]

                q = q0 + ci * _CQ
                g = q // _BB
                bb = lax.rem(q, _BB)
                pltpu.async_copy(tbuf[bp], out_hbm.at[g, :, pl.ds(bb, _CQ)],
                                 osem[bp])
        return carry

    lax.fori_loop(0, (_NCH + 2) // 3, step, None)
    for bp in range(3):
        pltpu.make_async_copy(tbuf[bp], out_hbm.at[0, :, pl.ds(0, _CQ)],
                              osem[bp]).wait()


_embed_call = functools.partial(
    pl.kernel,
    out_type=jax.ShapeDtypeStruct((_FIELDS, _FEAT // 8, _BB, 8, 128),
                                  jnp.float32),
    mesh=plsc.VectorSubcoreMesh(core_axis_name="c", subcore_axis_name="s"),
    scratch_types=[
        pltpu.VMEM((_LPW,), jnp.int32),
        pltpu.VMEM((_CL, _PF), jnp.float32),
        pltpu.VMEM((_CL, _PF), jnp.float32),
        pltpu.VMEM((_CL, _PF), jnp.float32),
        pltpu.VMEM((_FEAT // 8, _CQ, 8, 128), jnp.float32),
        pltpu.VMEM((_FEAT // 8, _CQ, 8, 128), jnp.float32),
        pltpu.VMEM((_FEAT // 8, _CQ, 8, 128), jnp.float32),
        pltpu.SemaphoreType.DMA,
        pltpu.SemaphoreType.DMA,
        pltpu.SemaphoreType.DMA,
        pltpu.SemaphoreType.DMA,
        pltpu.SemaphoreType.DMA,
        pltpu.SemaphoreType.DMA,
    ],
    compiler_params=pltpu.CompilerParams(
        use_tc_tiling_on_sc=False, needs_layout_passes=False),
)(_embed_body)


def kernel(inputs, embedding):
    idx = inputs.T.reshape(_NQ * 128).astype(jnp.int32)
    tail = jnp.pad(embedding[_FULLB * 128:].T, ((0, 0), (0, 128 - 64)))
    scratch = _table_call(embedding.T, tail)
    table = scratch.reshape(_VOCP, _PF)
    raw = _embed_call(idx, table)
    return raw.transpose(2, 4, 0, 1, 3).reshape(_BATCH, _FIELDS, _FEAT)


# speedup vs baseline: 1.0629x; 1.0629x over previous
import functools

import jax
import jax.numpy as jnp
from jax import lax
from jax.experimental import pallas as pl
from jax.experimental.pallas import tpu as pltpu
from jax.experimental.pallas import tpu_sc as plsc

_BATCH = 16384
_FIELDS = 26
_FEAT = 32
_PF = 40
_BB = _BATCH // 128
_NQ = _FIELDS * _BB
_NW = 32
_QPW = _NQ // _NW
_LPW = _QPW * 128
_CQ = 4
_CL = _CQ * 128
_NCH = _QPW // _CQ

_VOC = 1000000
_VOCP = 1000064
_NBLK = _VOCP // 128
_FULLB = _NBLK - 1
_SB = 4
_SBL = _SB * 128
_NSB = _FULLB // _SB
_SPW = _NSB // _NW
_SXTRA = _NSB - _SPW * _NW
_SCR = _VOCP * _PF


def _table_body(tt_hbm, tail_hbm, scr_hbm, in0, in1, in2, ob0, ob1, ob2,
                i0, i1, i2s, o0, o1, o2):
    c = lax.axis_index("c")
    s = lax.axis_index("s")
    wid = s * 2 + c
    start = wid * _SPW + jnp.minimum(wid, _SXTRA)
    nsb = _SPW + jnp.where(wid < _SXTRA, 1, 0)

    ibuf = (in0, in1, in2)
    obuf = (ob0, ob1, ob2)
    isem = (i0, i1, i2s)
    osem = (o0, o1, o2)
    iota16 = lax.iota(jnp.int32, 16)
    avecs = tuple((iota16 + v0) * _PF for v0 in range(0, _SBL, 16))

    def fetch(si, b):
        return pltpu.async_copy(
            tt_hbm.at[:, pl.ds((start + si) * _SBL, _SBL)], ibuf[b], isem[b])

    for p in range(2):
        fetch(p, p)

    def body(it, carry):
        for bp in range(3):
            si = 3 * it + bp

            @pl.when(si < nsb)
            def _():
                pltpu.make_async_copy(
                    tt_hbm.at[:, pl.ds(0, _SBL)], ibuf[bp], isem[bp]).wait()

                @pl.when(si + 2 < nsb)
                def _():
                    fetch(si + 2, (bp + 2) % 3)

                @pl.when(si >= 3)
                def _():
                    pltpu.make_async_copy(
                        obuf[bp], scr_hbm.at[pl.ds(0, _SBL * _PF)],
                        osem[bp]).wait()

                for f in range(_FEAT):
                    for v4 in range(0, _SBL // 16, 4):
                        vals = [ibuf[bp][f, pl.ds((v4 + u) * 16, 16)]
                                for u in range(4)]
                        for u in range(4):
                            plsc.store_scatter(obuf[bp], [avecs[v4 + u] + f],
                                               vals[u])

                pltpu.async_copy(
                    obuf[bp],
                    scr_hbm.at[pl.ds((start + si) * _SBL * _PF, _SBL * _PF)],
                    osem[bp])
        return carry

    lax.fori_loop(0, (_SPW + 3) // 3, body, None)
    for bp in range(3):
        pltpu.make_async_copy(
            obuf[bp], scr_hbm.at[pl.ds(0, _SBL * _PF)], osem[bp]).wait()

    @pl.when(wid == _NW - 1)
    def _():
        pltpu.sync_copy(tail_hbm, in0.at[:, pl.ds(0, 128)])
        for f in range(_FEAT):
            for u in range(4):
                vals = in0[f, pl.ds(u * 16, 16)]
                plsc.store_scatter(ob0, [avecs[u] + f], vals)
        pltpu.sync_copy(ob0.at[pl.ds(0, 64 * _PF)],
                        scr_hbm.at[pl.ds(_FULLB * 128 * _PF, 64 * _PF)])


_table_call = functools.partial(
    pl.kernel,
    out_type=jax.ShapeDtypeStruct((_SCR,), jnp.float32),
    mesh=plsc.VectorSubcoreMesh(core_axis_name="c", subcore_axis_name="s"),
    scratch_types=[
        pltpu.VMEM((_FEAT, _SBL), jnp.float32),
        pltpu.VMEM((_FEAT, _SBL), jnp.float32),
        pltpu.VMEM((_FEAT, _SBL), jnp.float32),
        pltpu.VMEM((_SBL * _PF,), jnp.float32),
        pltpu.VMEM((_SBL * _PF,), jnp.float32),
        pltpu.VMEM((_SBL * _PF,), jnp.float32),
        pltpu.SemaphoreType.DMA,
        pltpu.SemaphoreType.DMA,
        pltpu.SemaphoreType.DMA,
        pltpu.SemaphoreType.DMA,
        pltpu.SemaphoreType.DMA,
        pltpu.SemaphoreType.DMA,
    ],
    compiler_params=pltpu.CompilerParams(
        use_tc_tiling_on_sc=True, needs_layout_passes=False),
)(_table_body)


def _embed_body(idx_hbm, table_hbm, out_hbm, idx_v, rows0, rows1,
                t0, t1, g0, g1, o0, o1):
    c = lax.axis_index("c")
    s = lax.axis_index("s")
    wid = s * 2 + c
    q0 = wid * _QPW
    pltpu.sync_copy(idx_hbm.at[pl.ds(wid * _LPW, _LPW)], idx_v)

    rows = (rows0, rows1)
    tbuf = (t0, t1)
    gsem = (g0, g1)
    osem = (o0, o1)
    iota16 = lax.iota(jnp.int32, 16)
    fvecs = tuple(iota16 * 0 + f for f in range(_FEAT))

    def gather(ci, b):
        return pltpu.async_copy(
            table_hbm.at[idx_v.at[pl.ds(ci * _CL, _CL)]], rows[b], gsem[b])

    for p in range(2):
        gather(p, p)

    def step(it, carry):
        for bp in range(2):
            ci = 2 * it + bp

            @pl.when(ci < _NCH)
            def _():
                pltpu.make_async_copy(
                    table_hbm.at[idx_v.at[pl.ds(0, _CL)]], rows[bp],
                    gsem[bp]).wait()

                @pl.when(ci >= 2)
                def _():
                    pltpu.make_async_copy(
                        tbuf[bp], out_hbm.at[0, :, pl.ds(0, _CQ)],
                        osem[bp]).wait()

                for dq in range(_CQ):
                    for j in range(8):
                        kvec = iota16 + (dq * 128 + j * 16)
                        for f0 in range(0, _FEAT, 4):
                            vals = [plsc.load_gather(rows[bp],
                                                     [kvec, fvecs[f0 + u]])
                                    for u in range(4)]
                            for u in range(4):
                                f = f0 + u
                                tbuf[bp][f // 8, dq, f % 8,
                                         pl.ds(j * 16, 16)] = vals[u]

                @pl.when(ci + 2 < _NCH)
                def _():
                    gather(ci + 2, bp)

                q = q0 + ci * _CQ
                g = q // _BB
                bb = lax.rem(q, _BB)
                pltpu.async_copy(tbuf[bp], out_hbm.at[g, :, pl.ds(bb, _CQ)],
                                 osem[bp])
        return carry

    lax.fori_loop(0, _NCH // 2, step, None)
    for bp in range(2):
        pltpu.make_async_copy(tbuf[bp], out_hbm.at[0, :, pl.ds(0, _CQ)],
                              osem[bp]).wait()


_embed_call = functools.partial(
    pl.kernel,
    out_type=jax.ShapeDtypeStruct((_FIELDS, _FEAT // 8, _BB, 8, 128),
                                  jnp.float32),
    mesh=plsc.VectorSubcoreMesh(core_axis_name="c", subcore_axis_name="s"),
    scratch_types=[
        pltpu.VMEM((_LPW,), jnp.int32),
        pltpu.VMEM((_CL, _PF), jnp.float32),
        pltpu.VMEM((_CL, _PF), jnp.float32),
        pltpu.VMEM((_FEAT // 8, _CQ, 8, 128), jnp.float32),
        pltpu.VMEM((_FEAT // 8, _CQ, 8, 128), jnp.float32),
        pltpu.SemaphoreType.DMA,
        pltpu.SemaphoreType.DMA,
        pltpu.SemaphoreType.DMA,
        pltpu.SemaphoreType.DMA,
    ],
    compiler_params=pltpu.CompilerParams(
        use_tc_tiling_on_sc=False, needs_layout_passes=False),
)(_embed_body)


def kernel(inputs, embedding):
    idx = inputs.T.reshape(_NQ * 128).astype(jnp.int32)
    tail = jnp.pad(embedding[_FULLB * 128:].T, ((0, 0), (0, 128 - 64)))
    scratch = _table_call(embedding.T, tail)
    table = scratch.reshape(_VOCP, _PF)
    raw = _embed_call(idx, table)
    return raw.transpose(2, 4, 0, 1, 3).reshape(_BATCH, _FIELDS, _FEAT)


# speedup vs baseline: 1.1035x; 1.0382x over previous
import functools

import jax
import jax.numpy as jnp
from jax import lax
from jax.experimental import pallas as pl
from jax.experimental.pallas import tpu as pltpu
from jax.experimental.pallas import tpu_sc as plsc

_BATCH = 16384
_FIELDS = 26
_FEAT = 32
_PF = 40
_BB = _BATCH // 128
_NQ = _FIELDS * _BB
_NW = 32
_QPW = _NQ // _NW
_LPW = _QPW * 128
_CQ = 4
_CL = _CQ * 128
_NCH = _QPW // _CQ

_VOC = 1000000
_VOCP = 1000064
_NBLK = _VOCP // 128
_FULLB = _NBLK - 1
_SB = 4
_SBL = _SB * 128
_NSB = _FULLB // _SB
_SPW = _NSB // _NW
_SXTRA = _NSB - _SPW * _NW
_SCR = _VOCP * _PF


def _table_body(tt_hbm, tail_hbm, scr_hbm, in0, in1, ob0, ob1,
                i0, i1, o0, o1):
    c = lax.axis_index("c")
    s = lax.axis_index("s")
    wid = s * 2 + c
    start = wid * _SPW + jnp.minimum(wid, _SXTRA)
    nsb = _SPW + jnp.where(wid < _SXTRA, 1, 0)

    ibuf = (in0, in1)
    obuf = (ob0, ob1)
    isem = (i0, i1)
    osem = (o0, o1)
    iota16 = lax.iota(jnp.int32, 16)
    avecs = tuple((iota16 + v0) * _PF for v0 in range(0, _SBL, 16))

    def fetch(si, b):
        return pltpu.async_copy(
            tt_hbm.at[:, pl.ds((start + si) * _SBL, _SBL)], ibuf[b], isem[b])

    for p in range(2):
        fetch(p, p)

    def body(it, carry):
        for bp in range(2):
            si = 2 * it + bp

            @pl.when(si < nsb)
            def _():
                pltpu.make_async_copy(
                    tt_hbm.at[:, pl.ds(0, _SBL)], ibuf[bp], isem[bp]).wait()

                @pl.when(si >= 2)
                def _():
                    pltpu.make_async_copy(
                        obuf[bp], scr_hbm.at[pl.ds(0, _SBL * _PF)],
                        osem[bp]).wait()

                for f in range(_FEAT):
                    for v4 in range(0, _SBL // 16, 4):
                        vals = [ibuf[bp][f, pl.ds((v4 + u) * 16, 16)]
                                for u in range(4)]
                        for u in range(4):
                            plsc.store_scatter(obuf[bp], [avecs[v4 + u] + f],
                                               vals[u])

                @pl.when(si + 2 < nsb)
                def _():
                    fetch(si + 2, bp)

                pltpu.async_copy(
                    obuf[bp],
                    scr_hbm.at[pl.ds((start + si) * _SBL * _PF, _SBL * _PF)],
                    osem[bp])
        return carry

    lax.fori_loop(0, (_SPW + 2) // 2, body, None)
    for bp in range(2):
        pltpu.make_async_copy(
            obuf[bp], scr_hbm.at[pl.ds(0, _SBL * _PF)], osem[bp]).wait()

    @pl.when(wid == _NW - 1)
    def _():
        pltpu.sync_copy(tail_hbm, in0.at[:, pl.ds(0, 128)])
        for f in range(_FEAT):
            for u in range(4):
                vals = in0[f, pl.ds(u * 16, 16)]
                plsc.store_scatter(ob0, [avecs[u] + f], vals)
        pltpu.sync_copy(ob0.at[pl.ds(0, 64 * _PF)],
                        scr_hbm.at[pl.ds(_FULLB * 128 * _PF, 64 * _PF)])


_table_call = functools.partial(
    pl.kernel,
    out_type=jax.ShapeDtypeStruct((_SCR,), jnp.float32),
    mesh=plsc.VectorSubcoreMesh(core_axis_name="c", subcore_axis_name="s"),
    scratch_types=[
        pltpu.VMEM((_FEAT, _SBL), jnp.float32),
        pltpu.VMEM((_FEAT, _SBL), jnp.float32),
        pltpu.VMEM((_SBL * _PF,), jnp.float32),
        pltpu.VMEM((_SBL * _PF,), jnp.float32),
        pltpu.SemaphoreType.DMA,
        pltpu.SemaphoreType.DMA,
        pltpu.SemaphoreType.DMA,
        pltpu.SemaphoreType.DMA,
    ],
    compiler_params=pltpu.CompilerParams(
        use_tc_tiling_on_sc=True, needs_layout_passes=False),
)(_table_body)


def _embed_body(idx_hbm, table_hbm, out_hbm, idx_v, rows0, rows1,
                t0, t1, g0, g1, o0, o1):
    c = lax.axis_index("c")
    s = lax.axis_index("s")
    wid = s * 2 + c
    q0 = wid * _QPW
    pltpu.sync_copy(idx_hbm.at[pl.ds(wid * _LPW, _LPW)], idx_v)

    rows = (rows0, rows1)
    tbuf = (t0, t1)
    gsem = (g0, g1)
    osem = (o0, o1)
    iota16 = lax.iota(jnp.int32, 16)
    fvecs = tuple(iota16 * 0 + f for f in range(_FEAT))

    def gather(ci, b):
        return pltpu.async_copy(
            table_hbm.at[idx_v.at[pl.ds(ci * _CL, _CL)]], rows[b], gsem[b])

    for p in range(2):
        gather(p, p)

    def step(it, carry):
        for bp in range(2):
            ci = 2 * it + bp

            @pl.when(ci < _NCH)
            def _():
                pltpu.make_async_copy(
                    table_hbm.at[idx_v.at[pl.ds(0, _CL)]], rows[bp],
                    gsem[bp]).wait()

                @pl.when(ci >= 2)
                def _():
                    pltpu.make_async_copy(
                        tbuf[bp], out_hbm.at[0, :, pl.ds(0, _CQ)],
                        osem[bp]).wait()

                for dq in range(_CQ):
                    for j in range(8):
                        kvec = iota16 + (dq * 128 + j * 16)
                        for f0 in range(0, _FEAT, 4):
                            vals = [plsc.load_gather(rows[bp],
                                                     [kvec, fvecs[f0 + u]])
                                    for u in range(4)]
                            for u in range(4):
                                f = f0 + u
                                tbuf[bp][f // 8, dq, f % 8,
                                         pl.ds(j * 16, 16)] = vals[u]

                @pl.when(ci + 2 < _NCH)
                def _():
                    gather(ci + 2, bp)

                q = q0 + ci * _CQ
                g = q // _BB
                bb = lax.rem(q, _BB)
                pltpu.async_copy(tbuf[bp], out_hbm.at[g, :, pl.ds(bb, _CQ)],
                                 osem[bp])
        return carry

    lax.fori_loop(0, _NCH // 2, step, None)
    for bp in range(2):
        pltpu.make_async_copy(tbuf[bp], out_hbm.at[0, :, pl.ds(0, _CQ)],
                              osem[bp]).wait()


_embed_call = functools.partial(
    pl.kernel,
    out_type=jax.ShapeDtypeStruct((_FIELDS, _FEAT // 8, _BB, 8, 128),
                                  jnp.float32),
    mesh=plsc.VectorSubcoreMesh(core_axis_name="c", subcore_axis_name="s"),
    scratch_types=[
        pltpu.VMEM((_LPW,), jnp.int32),
        pltpu.VMEM((_CL, _PF), jnp.float32),
        pltpu.VMEM((_CL, _PF), jnp.float32),
        pltpu.VMEM((_FEAT // 8, _CQ, 8, 128), jnp.float32),
        pltpu.VMEM((_FEAT // 8, _CQ, 8, 128), jnp.float32),
        pltpu.SemaphoreType.DMA,
        pltpu.SemaphoreType.DMA,
        pltpu.SemaphoreType.DMA,
        pltpu.SemaphoreType.DMA,
    ],
    compiler_params=pltpu.CompilerParams(
        use_tc_tiling_on_sc=False, needs_layout_passes=False),
)(_embed_body)


def kernel(inputs, embedding):
    idx = inputs.T.reshape(_NQ * 128).astype(jnp.int32)
    tail = jnp.pad(embedding[_FULLB * 128:].T, ((0, 0), (0, 128 - 64)))
    scratch = _table_call(embedding.T, tail)
    table = scratch.reshape(_VOCP, _PF)
    raw = _embed_call(idx, table)
    return raw.transpose(2, 4, 0, 1, 3).reshape(_BATCH, _FIELDS, _FEAT)
